# baseline (speedup 1.0000x reference)
import functools

import jax
import jax.numpy as jnp
from jax import lax
from jax.experimental import pallas as pl
from jax.experimental.pallas import tpu as pltpu
from jax.experimental.pallas import tpu_sc as plsc

_B = 1024
_D = 64
_NH = 4
_DH = 16
_LL = 200
_LS = 50
_TOPK = 50
_NB = 1024

_NW = 32
_CHUNK = 128
_TOT = _B * _LL + _B * _LS + _B
_NCH = 64
_PAD_TOT = _NW * _NCH * _CHUNK


def _sc_gather(table, ids):
    mesh = plsc.VectorSubcoreMesh(core_axis_name="c", subcore_axis_name="s")

    @functools.partial(
        pl.kernel,
        mesh=mesh,
        compiler_params=pltpu.CompilerParams(use_tc_tiling_on_sc=False),
        out_type=jax.ShapeDtypeStruct((_PAD_TOT, _D), jnp.float32),
        scratch_types=[
            pltpu.VMEM((_NCH, _CHUNK), jnp.int32),
            pltpu.VMEM((4, _CHUNK, _D), jnp.float32),
            pltpu.SemaphoreType.DMA,
            pltpu.SemaphoreType.DMA,
        ],
    )
    def k(table_hbm, ids_hbm, out_hbm, idx_v, rows_v, semg, semw):
        wid = lax.axis_index("s") * 2 + lax.axis_index("c")
        base = wid * (_NCH * _CHUNK)
        pltpu.sync_copy(ids_hbm.at[wid], idx_v)
        for j in range(3):
            pltpu.async_copy(table_hbm.at[idx_v.at[j]], rows_v.at[j], semg)

        def body(i, _):
            slot = lax.rem(i, 4)
            pltpu.make_async_copy(
                table_hbm.at[idx_v.at[i]], rows_v.at[slot], semg).wait()
            pltpu.async_copy(
                rows_v.at[slot], out_hbm.at[pl.ds(base + i * _CHUNK, _CHUNK)],
                semw)

            @pl.when(i + 3 < _NCH)
            def _start():
                nslot = lax.rem(i + 3, 4)

                @pl.when(i >= 1)
                def _drain():
                    pltpu.make_async_copy(
                        rows_v.at[nslot],
                        out_hbm.at[pl.ds(base + i * _CHUNK, _CHUNK)],
                        semw).wait()

                pltpu.async_copy(
                    table_hbm.at[idx_v.at[i + 3]], rows_v.at[nslot], semg)

            return _

        lax.fori_loop(0, _NCH, body, None)
        def drain(i, _):
            pltpu.make_async_copy(
                rows_v.at[0], out_hbm.at[pl.ds(base, _CHUNK)], semw).wait()
            return _

        lax.fori_loop(0, 4, drain, None)

    return k(table, ids)


def _layernorm(x, g, b, eps=1e-5):
    m = jnp.mean(x, axis=-1, keepdims=True)
    v = jnp.mean((x - m) ** 2, axis=-1, keepdims=True)
    return (x - m) / jnp.sqrt(v + eps) * g + b


def _make_tc_body(bb):
    n_tok = bb * _LS

    def body(uf_ref, le_ref, xs_ref, tr_ref, idm_ref,
             wt, bt, wa1t, wa1l, wa1x, ba1, wa2, ba2,
             hcat, wsp, bsp, lyr0, lyr1,
             wfa, bfa, wfb, bfb, wfc, bfc, out_ref):
        uf = uf_ref[...]
        le = le_ref[...]
        xs = xs_ref[...]
        tr = tr_ref[...]
        idm = idm_ref[...]

        te = jnp.dot(tr, wt[...]) + bt[...]

        le2 = le.reshape(bb * _LL, _D)
        tex = jnp.broadcast_to(te[:, None, :], (bb, _LL, _D)).reshape(bb * _LL, _D)
        pre = jnp.dot(te, wa1t[...]) + ba1[...]
        preb = jnp.broadcast_to(pre[:, None, :], (bb, _LL, _D)).reshape(bb * _LL, _D)
        h1 = jax.nn.relu(jnp.dot(le2, wa1l[...]) + jnp.dot(le2 * tex, wa1x[...]) + preb)
        s = (jnp.dot(h1, wa2[...]) + ba2[...]).reshape(bb, _LL)

        mx = jnp.max(s, axis=1, keepdims=True)
        e = jnp.exp(s - mx)
        aw = e / jnp.sum(e, axis=1, keepdims=True)

        bits = lax.bitcast_convert_type(aw, jnp.int32)
        lo0 = jnp.zeros((bb, 1), jnp.int32)
        hi0 = jnp.full((bb, 1), 0x3F800001, jnp.int32)

        def bs(_, c):
            lo, hi = c
            mid = (lo + hi) >> 1
            cnt = jnp.sum(jnp.where(bits >= mid, 1.0, 0.0), axis=1, keepdims=True)
            p = cnt >= float(_TOPK)
            return jnp.where(p, mid, lo), jnp.where(p, hi, mid)

        tlo, _ = lax.fori_loop(0, 31, bs, (lo0, hi0))
        gt = bits > tlo
        eq = bits == tlo
        c_gt = jnp.sum(jnp.where(gt, 1.0, 0.0), axis=1, keepdims=True)
        need = float(_TOPK) - c_gt
        ii = lax.broadcasted_iota(jnp.int32, (_LL, _LL), 0)
        jj = lax.broadcasted_iota(jnp.int32, (_LL, _LL), 1)
        ltri = jnp.where(ii <= jj, 1.0, 0.0)
        prefix = jnp.dot(jnp.where(eq, 1.0, 0.0), ltri)
        sel = gt | (eq & (prefix <= need))

        awm = jnp.max(aw, axis=1, keepdims=True)
        w = jnp.where(sel, jnp.exp(aw - awm), 0.0)
        wsum = jnp.sum(w, axis=1, keepdims=True)
        hard = jnp.sum((w / wsum)[:, :, None] * le, axis=1)
        lemean = jnp.mean(le, axis=1)

        iot = lax.broadcasted_iota(jnp.int16, (bb, _LL, _NB), 2)
        oneh = jnp.where(idm.astype(jnp.int16)[:, :, None] == iot,
                         jnp.bfloat16(1.0), jnp.bfloat16(0.0))
        obd = jnp.where(
            lax.broadcasted_iota(jnp.int32, (bb, bb * _LL), 0)
            == lax.broadcasted_iota(jnp.int32, (bb, bb * _LL), 1) // _LL,
            1.0, 0.0).astype(jnp.bfloat16)
        counts = jnp.dot(obd, oneh.reshape(bb * _LL, _NB),
                         preferred_element_type=jnp.float32)
        hr = jnp.dot(counts * (1.0 / _LL), hcat[...])
        soft = jnp.dot(hr + lemean, wsp[...]) + bsp[...]

        x = xs.reshape(n_tok, _D)
        bf = jnp.bfloat16
        x = x.astype(bf)
        hmask = jnp.where(
            lax.broadcasted_iota(jnp.int32, (_NH * _LS, _D), 0) // _LS
            == lax.broadcasted_iota(jnp.int32, (_NH * _LS, _D), 1) // _DH,
            1.0, 0.0).astype(bf)

        for (wqkv, bqkv, wo, bo, g1, b1, w1, bff1, w2, bff2, g2, b2) in (lyr0, lyr1):
            qkv = (jnp.dot(x, wqkv[...].astype(bf), preferred_element_type=jnp.float32)
                   + bqkv[...]).astype(bf)
            vbigs, scs = [], []
            for smp in range(bb):
                sl = slice(smp * _LS, (smp + 1) * _LS)
                q_s = qkv[sl, :_D]
                k_s = qkv[sl, _D:2 * _D]
                v_s = qkv[sl, 2 * _D:]
                kbig = jnp.concatenate([k_s] * _NH, axis=0) * hmask
                vbigs.append(jnp.concatenate([v_s] * _NH, axis=0) * hmask)
                scs.append(lax.dot_general(
                    q_s, kbig, (((1,), (1,)), ((), ())),
                    preferred_element_type=jnp.float32).astype(bf))
            sall = jnp.concatenate(scs, axis=0) * bf(0.25)
            aparts = []
            for h in range(_NH):
                blk = sall[:, h * _LS:(h + 1) * _LS]
                bm = jnp.max(blk, axis=1, keepdims=True)
                ebk = jnp.exp(blk - bm)
                aparts.append(ebk / jnp.sum(ebk, axis=1, keepdims=True))
            aall = jnp.concatenate(aparts, axis=1)
            outs = [
                jnp.dot(aall[smp * _LS:(smp + 1) * _LS], vbigs[smp],
                        preferred_element_type=jnp.float32).astype(bf)
                for smp in range(bb)
            ]
            o = (jnp.dot(jnp.concatenate(outs, axis=0),
                         wo[...].astype(bf),
                         preferred_element_type=jnp.float32).astype(bf)
                 + bo[...].astype(bf))
            x = _layernorm(x + o, g1[...].astype(bf), b1[...].astype(bf))
            ff = (jnp.dot(
                jax.nn.relu(
                    jnp.dot(x, w1[...].astype(bf),
                            preferred_element_type=jnp.float32).astype(bf)
                    + bff1[...].astype(bf)),
                w2[...].astype(bf),
                preferred_element_type=jnp.float32).astype(bf)
                + bff2[...].astype(bf))
            x = _layernorm(x + ff, g2[...].astype(bf), b2[...].astype(bf))

        bst = jnp.mean(x.astype(jnp.float32).reshape(bb, _LS, _D), axis=1)

        comb = jnp.concatenate([uf, hard, soft, bst], axis=1)
        hh = jax.nn.relu(jnp.dot(comb, wfa[...]) + bfa[...])
        hh = jax.nn.relu(jnp.dot(hh, wfb[...]) + bfb[...])
        out_ref[...] = jnp.dot(hh, wfc[...]) + bfc[...]

    return body


def _tc_forward(uf, le, xs, tr, idm, p, bb=64, interpret=False):
    row = lambda v: v.reshape(1, -1)
    hcat = jnp.concatenate([p['hash0'], p['hash1'], p['hash2'], p['hash3']], axis=1)
    wa1 = p['Wa1']
    lyrs = []
    for i in range(2):
        lyrs.append([
            p['Wqkv%d' % i], row(p['bqkv%d' % i]), p['Wo%d' % i], row(p['bo%d' % i]),
            row(p['g1_%d' % i]), row(p['b1_%d' % i]), p['W1_%d' % i], row(p['bff1_%d' % i]),
            p['W2_%d' % i], row(p['bff2_%d' % i]), row(p['g2_%d' % i]), row(p['b2_%d' % i]),
        ])
    weights = [
        p['Wt'], row(p['bt']),
        wa1[:_D], wa1[_D:2 * _D], wa1[2 * _D:], row(p['ba1']),
        p['Wa2'], row(p['ba2']),
        hcat, p['Wsp'], row(p['bsp']),
        lyrs[0], lyrs[1],
        p['Wfa'], row(p['bfa']), p['Wfb'], row(p['bfb']), p['Wfc'], row(p['bfc']),
    ]

    def wspec(w):
        nd = w.ndim
        return pl.BlockSpec(w.shape, lambda i, _n=nd: (0,) * _n)

    in_specs = [
        pl.BlockSpec((bb, _D), lambda i: (i, 0)),
        pl.BlockSpec((bb, _LL, _D), lambda i: (i, 0, 0)),
        pl.BlockSpec((bb, _LS, _D), lambda i: (i, 0, 0)),
        pl.BlockSpec((bb, _D), lambda i: (i, 0)),
        pl.BlockSpec((bb, _LL), lambda i: (i, 0)),
    ] + jax.tree.map(wspec, weights)

    out = pl.pallas_call(
        _make_tc_body(bb),
        grid=(_B // bb,),
        in_specs=in_specs,
        out_specs=pl.BlockSpec((bb, 1), lambda i: (i, 0)),
        out_shape=jax.ShapeDtypeStruct((_B, 1), jnp.float32),
        interpret=interpret,
    )(uf, le, xs, tr, idm, *weights)
    return out[:, 0]


def kernel(user_features, params, target_item_id, short_hist_ids, long_hist_ids):
    ids_all = jnp.concatenate([
        long_hist_ids.reshape(-1),
        short_hist_ids.reshape(-1),
        target_item_id,
        jnp.zeros((_PAD_TOT - _TOT,), jnp.int32),
    ]).astype(jnp.int32).reshape(_NW, _NCH, _CHUNK)
    rows = _sc_gather(params['table'], ids_all)
    le = rows[:_B * _LL].reshape(_B, _LL, _D)
    xs = rows[_B * _LL:_B * _LL + _B * _LS].reshape(_B, _LS, _D)
    tr = rows[_B * _LL + _B * _LS:_TOT]
    idm = (long_hist_ids % _NB).astype(jnp.int32)
    return _tc_forward(user_features, le, xs, tr, idm, params)


# speedup vs baseline: 1.0188x; 1.0188x over previous
import functools

import jax
import jax.numpy as jnp
from jax import lax
from jax.experimental import pallas as pl
from jax.experimental.pallas import tpu as pltpu
from jax.experimental.pallas import tpu_sc as plsc

_B = 1024
_D = 64
_NH = 4
_DH = 16
_LL = 200
_LS = 50
_TOPK = 50
_NB = 1024

_NW = 32
_CHUNK = 128
_TOT = _B * _LL + _B * _LS + _B
_TABLE_ROWS = 1000001
_NCH = 64
_PAD_TOT = _NW * _NCH * _CHUNK


def _sc_gather(table, ids):
    mesh = plsc.VectorSubcoreMesh(core_axis_name="c", subcore_axis_name="s")

    @functools.partial(
        pl.kernel,
        mesh=mesh,
        compiler_params=pltpu.CompilerParams(use_tc_tiling_on_sc=False),
        out_type=jax.ShapeDtypeStruct((_PAD_TOT, _D), jnp.float32),
        scratch_types=[
            pltpu.VMEM((_NCH, _CHUNK), jnp.int32),
            pltpu.VMEM((4, _CHUNK, _D), jnp.float32),
            pltpu.SemaphoreType.DMA,
            pltpu.SemaphoreType.DMA,
        ],
    )
    def k(table_hbm, ids_hbm, out_hbm, idx_v, rows_v, semg, semw):
        wid = lax.axis_index("s") * 2 + lax.axis_index("c")
        base = wid * (_NCH * _CHUNK)
        pltpu.sync_copy(ids_hbm.at[wid], idx_v)
        for j in range(3):
            pltpu.async_copy(table_hbm.at[idx_v.at[j]], rows_v.at[j], semg)

        def body(i, _):
            slot = lax.rem(i, 4)
            pltpu.make_async_copy(
                table_hbm.at[idx_v.at[i]], rows_v.at[slot], semg).wait()
            pltpu.async_copy(
                rows_v.at[slot], out_hbm.at[pl.ds(base + i * _CHUNK, _CHUNK)],
                semw)

            @pl.when(i + 3 < _NCH)
            def _start():
                nslot = lax.rem(i + 3, 4)

                @pl.when(i >= 1)
                def _drain():
                    pltpu.make_async_copy(
                        rows_v.at[nslot],
                        out_hbm.at[pl.ds(base + i * _CHUNK, _CHUNK)],
                        semw).wait()

                pltpu.async_copy(
                    table_hbm.at[idx_v.at[i + 3]], rows_v.at[nslot], semg)

            return _

        lax.fori_loop(0, _NCH, body, None)
        def drain(i, _):
            pltpu.make_async_copy(
                rows_v.at[0], out_hbm.at[pl.ds(base, _CHUNK)], semw).wait()
            return _

        lax.fori_loop(0, 4, drain, None)

    return k(table, ids)


def _layernorm(x, g, b, eps=1e-5):
    m = jnp.mean(x, axis=-1, keepdims=True)
    v = jnp.mean((x - m) ** 2, axis=-1, keepdims=True)
    return (x - m) / jnp.sqrt(v + eps) * g + b


def _make_tc_body(bb):
    n_tok = bb * _LS

    def body(uf_ref, le_ref, xs_ref, tr_ref, idm_ref,
             wt, bt, wa1t, wa1l, wa1x, ba1, wa2, ba2,
             hcat, wsp, bsp, lyr0, lyr1,
             wfa, bfa, wfb, bfb, wfc, bfc, out_ref):
        uf = uf_ref[...]
        le = le_ref[...]
        xs = xs_ref[...]
        tr = tr_ref[...]
        idm = idm_ref[...]

        te = jnp.dot(tr, wt[...]) + bt[...]

        le2 = le.reshape(bb * _LL, _D)
        tex = jnp.broadcast_to(te[:, None, :], (bb, _LL, _D)).reshape(bb * _LL, _D)
        pre = jnp.dot(te, wa1t[...]) + ba1[...]
        preb = jnp.broadcast_to(pre[:, None, :], (bb, _LL, _D)).reshape(bb * _LL, _D)
        h1 = jax.nn.relu(jnp.dot(le2, wa1l[...]) + jnp.dot(le2 * tex, wa1x[...]) + preb)
        s = (jnp.dot(h1, wa2[...]) + ba2[...]).reshape(bb, _LL)

        mx = jnp.max(s, axis=1, keepdims=True)
        e = jnp.exp(s - mx)
        aw = e / jnp.sum(e, axis=1, keepdims=True)

        bits = lax.bitcast_convert_type(aw, jnp.int32)
        lo0 = jnp.zeros((bb, 1), jnp.int32)
        hi0 = jnp.full((bb, 1), 0x3F800001, jnp.int32)

        def bs(_, c):
            lo, hi = c
            mid = (lo + hi) >> 1
            cnt = jnp.sum(jnp.where(bits >= mid, 1.0, 0.0), axis=1, keepdims=True)
            p = cnt >= float(_TOPK)
            return jnp.where(p, mid, lo), jnp.where(p, hi, mid)

        tlo, _ = lax.fori_loop(0, 31, bs, (lo0, hi0))
        gt = bits > tlo
        eq = bits == tlo
        c_gt = jnp.sum(jnp.where(gt, 1.0, 0.0), axis=1, keepdims=True)
        need = float(_TOPK) - c_gt
        ii = lax.broadcasted_iota(jnp.int32, (_LL, _LL), 0)
        jj = lax.broadcasted_iota(jnp.int32, (_LL, _LL), 1)
        ltri = jnp.where(ii <= jj, 1.0, 0.0)
        prefix = jnp.dot(jnp.where(eq, 1.0, 0.0), ltri)
        sel = gt | (eq & (prefix <= need))

        awm = jnp.max(aw, axis=1, keepdims=True)
        w = jnp.where(sel, jnp.exp(aw - awm), 0.0)
        wsum = jnp.sum(w, axis=1, keepdims=True)
        hard = jnp.sum((w / wsum)[:, :, None] * le, axis=1)
        lemean = jnp.mean(le, axis=1)

        iot = lax.broadcasted_iota(jnp.int16, (bb, _LL, _NB), 2)
        oneh = jnp.where(idm.astype(jnp.int16)[:, :, None] == iot,
                         jnp.bfloat16(1.0), jnp.bfloat16(0.0))
        obd = jnp.where(
            lax.broadcasted_iota(jnp.int32, (bb, bb * _LL), 0)
            == lax.broadcasted_iota(jnp.int32, (bb, bb * _LL), 1) // _LL,
            1.0, 0.0).astype(jnp.bfloat16)
        counts = jnp.dot(obd, oneh.reshape(bb * _LL, _NB),
                         preferred_element_type=jnp.float32)
        hr = jnp.dot(counts * (1.0 / _LL), hcat[...])
        soft = jnp.dot(hr + lemean, wsp[...]) + bsp[...]

        x = xs.reshape(n_tok, _D)
        bf = jnp.bfloat16
        x = x.astype(bf)
        hmask = jnp.where(
            lax.broadcasted_iota(jnp.int32, (_NH * _LS, _D), 0) // _LS
            == lax.broadcasted_iota(jnp.int32, (_NH * _LS, _D), 1) // _DH,
            1.0, 0.0).astype(bf)

        for (wqkv, bqkv, wo, bo, g1, b1, w1, bff1, w2, bff2, g2, b2) in (lyr0, lyr1):
            qkv = (jnp.dot(x, wqkv[...].astype(bf), preferred_element_type=jnp.float32)
                   + bqkv[...]).astype(bf)
            vbigs, scs = [], []
            for smp in range(bb):
                sl = slice(smp * _LS, (smp + 1) * _LS)
                q_s = qkv[sl, :_D]
                k_s = qkv[sl, _D:2 * _D]
                v_s = qkv[sl, 2 * _D:]
                kbig = jnp.concatenate([k_s] * _NH, axis=0) * hmask
                vbigs.append(jnp.concatenate([v_s] * _NH, axis=0) * hmask)
                scs.append(lax.dot_general(
                    q_s, kbig, (((1,), (1,)), ((), ())),
                    preferred_element_type=jnp.float32).astype(bf))
            sall = jnp.concatenate(scs, axis=0) * bf(0.25)
            aparts = []
            for h in range(_NH):
                blk = sall[:, h * _LS:(h + 1) * _LS]
                bm = jnp.max(blk, axis=1, keepdims=True)
                ebk = jnp.exp(blk - bm)
                aparts.append(ebk / jnp.sum(ebk, axis=1, keepdims=True))
            aall = jnp.concatenate(aparts, axis=1)
            outs = [
                jnp.dot(aall[smp * _LS:(smp + 1) * _LS], vbigs[smp],
                        preferred_element_type=jnp.float32).astype(bf)
                for smp in range(bb)
            ]
            o = (jnp.dot(jnp.concatenate(outs, axis=0),
                         wo[...].astype(bf),
                         preferred_element_type=jnp.float32).astype(bf)
                 + bo[...].astype(bf))
            x = _layernorm(x + o, g1[...].astype(bf), b1[...].astype(bf))
            ff = (jnp.dot(
                jax.nn.relu(
                    jnp.dot(x, w1[...].astype(bf),
                            preferred_element_type=jnp.float32).astype(bf)
                    + bff1[...].astype(bf)),
                w2[...].astype(bf),
                preferred_element_type=jnp.float32).astype(bf)
                + bff2[...].astype(bf))
            x = _layernorm(x + ff, g2[...].astype(bf), b2[...].astype(bf))

        bst = jnp.mean(x.astype(jnp.float32).reshape(bb, _LS, _D), axis=1)

        comb = jnp.concatenate([uf, hard, soft, bst], axis=1)
        hh = jax.nn.relu(jnp.dot(comb, wfa[...]) + bfa[...])
        hh = jax.nn.relu(jnp.dot(hh, wfb[...]) + bfb[...])
        out_ref[...] = jnp.dot(hh, wfc[...]) + bfc[...]

    return body


def _tc_forward(uf, le, xs, tr, idm, p, bb=32, interpret=False):
    row = lambda v: v.reshape(1, -1)
    hcat = jnp.concatenate([p['hash0'], p['hash1'], p['hash2'], p['hash3']], axis=1)
    wa1 = p['Wa1']
    lyrs = []
    for i in range(2):
        lyrs.append([
            p['Wqkv%d' % i], row(p['bqkv%d' % i]), p['Wo%d' % i], row(p['bo%d' % i]),
            row(p['g1_%d' % i]), row(p['b1_%d' % i]), p['W1_%d' % i], row(p['bff1_%d' % i]),
            p['W2_%d' % i], row(p['bff2_%d' % i]), row(p['g2_%d' % i]), row(p['b2_%d' % i]),
        ])
    weights = [
        p['Wt'], row(p['bt']),
        wa1[:_D], wa1[_D:2 * _D], wa1[2 * _D:], row(p['ba1']),
        p['Wa2'], row(p['ba2']),
        hcat, p['Wsp'], row(p['bsp']),
        lyrs[0], lyrs[1],
        p['Wfa'], row(p['bfa']), p['Wfb'], row(p['bfb']), p['Wfc'], row(p['bfc']),
    ]

    def wspec(w):
        nd = w.ndim
        return pl.BlockSpec(w.shape, lambda i, _n=nd: (0,) * _n)

    in_specs = [
        pl.BlockSpec((bb, _D), lambda i: (i, 0)),
        pl.BlockSpec((bb, _LL, _D), lambda i: (i, 0, 0)),
        pl.BlockSpec((bb, _LS, _D), lambda i: (i, 0, 0)),
        pl.BlockSpec((bb, _D), lambda i: (i, 0)),
        pl.BlockSpec((bb, _LL), lambda i: (i, 0)),
    ] + jax.tree.map(wspec, weights)

    out = pl.pallas_call(
        _make_tc_body(bb),
        grid=(_B // bb,),
        in_specs=in_specs,
        out_specs=pl.BlockSpec((bb, 1), lambda i: (i, 0)),
        out_shape=jax.ShapeDtypeStruct((_B, 1), jnp.float32),
        interpret=interpret,
    )(uf, le, xs, tr, idm, *weights)
    return out[:, 0]


def kernel(user_features, params, target_item_id, short_hist_ids, long_hist_ids):
    ids_all = jnp.concatenate([
        long_hist_ids.reshape(-1),
        short_hist_ids.reshape(-1),
        target_item_id,
        jnp.zeros((_PAD_TOT - _TOT,), jnp.int32),
    ]).astype(jnp.int32).reshape(_NW, _NCH, _CHUNK)
    tflat = lax.optimization_barrier(params['table'].reshape(-1))
    rows = _sc_gather(tflat.reshape(_TABLE_ROWS, _D), ids_all)
    le = rows[:_B * _LL].reshape(_B, _LL, _D)
    xs = rows[_B * _LL:_B * _LL + _B * _LS].reshape(_B, _LS, _D)
    tr = rows[_B * _LL + _B * _LS:_TOT]
    idm = (long_hist_ids % _NB).astype(jnp.int32)
    return _tc_forward(user_features, le, xs, tr, idm, params)


# speedup vs baseline: 1.0195x; 1.0007x over previous
import functools

import jax
import jax.numpy as jnp
from jax import lax
from jax.experimental import pallas as pl
from jax.experimental.pallas import tpu as pltpu
from jax.experimental.pallas import tpu_sc as plsc

_B = 1024
_D = 64
_NH = 4
_DH = 16
_LL = 200
_LS = 50
_TOPK = 50
_NB = 1024

_NW = 32
_CHUNK = 128
_TOT = _B * _LL + _B * _LS + _B
_NCH = 64
_PAD_TOT = _NW * _NCH * _CHUNK


def _sc_gather(table, ids):
    mesh = plsc.VectorSubcoreMesh(core_axis_name="c", subcore_axis_name="s")

    @functools.partial(
        pl.kernel,
        mesh=mesh,
        compiler_params=pltpu.CompilerParams(use_tc_tiling_on_sc=False),
        out_type=jax.ShapeDtypeStruct((_PAD_TOT, _D), jnp.float32),
        scratch_types=[
            pltpu.VMEM((_NCH, _CHUNK), jnp.int32),
            pltpu.VMEM((4, _CHUNK, _D), jnp.float32),
            pltpu.SemaphoreType.DMA,
            pltpu.SemaphoreType.DMA,
        ],
    )
    def k(table_hbm, ids_hbm, out_hbm, idx_v, rows_v, semg, semw):
        wid = lax.axis_index("s") * 2 + lax.axis_index("c")
        base = wid * (_NCH * _CHUNK)
        pltpu.sync_copy(ids_hbm.at[wid], idx_v)
        for j in range(3):
            pltpu.async_copy(table_hbm.at[idx_v.at[j]], rows_v.at[j], semg)

        def body(i, _):
            slot = lax.rem(i, 4)
            pltpu.make_async_copy(
                table_hbm.at[idx_v.at[i]], rows_v.at[slot], semg).wait()
            pltpu.async_copy(
                rows_v.at[slot], out_hbm.at[pl.ds(base + i * _CHUNK, _CHUNK)],
                semw)

            @pl.when(i + 3 < _NCH)
            def _start():
                nslot = lax.rem(i + 3, 4)

                @pl.when(i >= 1)
                def _drain():
                    pltpu.make_async_copy(
                        rows_v.at[nslot],
                        out_hbm.at[pl.ds(base + i * _CHUNK, _CHUNK)],
                        semw).wait()

                pltpu.async_copy(
                    table_hbm.at[idx_v.at[i + 3]], rows_v.at[nslot], semg)

            return _

        lax.fori_loop(0, _NCH, body, None)
        def drain(i, _):
            pltpu.make_async_copy(
                rows_v.at[0], out_hbm.at[pl.ds(base, _CHUNK)], semw).wait()
            return _

        lax.fori_loop(0, 4, drain, None)

    return k(table, ids)


def _layernorm(x, g, b, eps=1e-5):
    m = jnp.mean(x, axis=-1, keepdims=True)
    v = jnp.mean((x - m) ** 2, axis=-1, keepdims=True)
    return (x - m) / jnp.sqrt(v + eps) * g + b


def _make_tc_body(bb):
    n_tok = bb * _LS

    def body(uf_ref, le_ref, xs_ref, tr_ref, idm_ref,
             wt, bt, wa1t, wa1l, wa1x, ba1, wa2, ba2,
             hcat, wsp, bsp, lyr0, lyr1,
             wfa, bfa, wfb, bfb, wfc, bfc, out_ref):
        uf = uf_ref[...]
        le = le_ref[...]
        xs = xs_ref[...]
        tr = tr_ref[...]
        idm = idm_ref[...]

        te = jnp.dot(tr, wt[...]) + bt[...]

        le2 = le.reshape(bb * _LL, _D)
        tex = jnp.broadcast_to(te[:, None, :], (bb, _LL, _D)).reshape(bb * _LL, _D)
        pre = jnp.dot(te, wa1t[...]) + ba1[...]
        preb = jnp.broadcast_to(pre[:, None, :], (bb, _LL, _D)).reshape(bb * _LL, _D)
        h1 = jax.nn.relu(jnp.dot(le2, wa1l[...]) + jnp.dot(le2 * tex, wa1x[...]) + preb)
        s = (jnp.dot(h1, wa2[...]) + ba2[...]).reshape(bb, _LL)

        mx = jnp.max(s, axis=1, keepdims=True)
        e = jnp.exp(s - mx)
        aw = e / jnp.sum(e, axis=1, keepdims=True)

        bits = lax.bitcast_convert_type(aw, jnp.int32)
        lo0 = jnp.zeros((bb, 1), jnp.int32)
        hi0 = jnp.full((bb, 1), 0x3F800001, jnp.int32)

        def bs(_, c):
            lo, hi = c
            mid = (lo + hi) >> 1
            cnt = jnp.sum(jnp.where(bits >= mid, 1.0, 0.0), axis=1, keepdims=True)
            p = cnt >= float(_TOPK)
            return jnp.where(p, mid, lo), jnp.where(p, hi, mid)

        tlo, _ = lax.fori_loop(0, 31, bs, (lo0, hi0))
        gt = bits > tlo
        eq = bits == tlo
        c_gt = jnp.sum(jnp.where(gt, 1.0, 0.0), axis=1, keepdims=True)
        need = float(_TOPK) - c_gt
        ii = lax.broadcasted_iota(jnp.int32, (_LL, _LL), 0)
        jj = lax.broadcasted_iota(jnp.int32, (_LL, _LL), 1)
        ltri = jnp.where(ii <= jj, 1.0, 0.0)
        prefix = jnp.dot(jnp.where(eq, 1.0, 0.0), ltri)
        sel = gt | (eq & (prefix <= need))

        awm = jnp.max(aw, axis=1, keepdims=True)
        w = jnp.where(sel, jnp.exp(aw - awm), 0.0)
        wsum = jnp.sum(w, axis=1, keepdims=True)
        hard = jnp.sum((w / wsum)[:, :, None] * le, axis=1)
        lemean = jnp.mean(le, axis=1)

        iot = lax.broadcasted_iota(jnp.int16, (bb, _LL, _NB), 2)
        oneh = jnp.where(idm.astype(jnp.int16)[:, :, None] == iot,
                         jnp.bfloat16(1.0), jnp.bfloat16(0.0))
        obd = jnp.where(
            lax.broadcasted_iota(jnp.int32, (bb, bb * _LL), 0)
            == lax.broadcasted_iota(jnp.int32, (bb, bb * _LL), 1) // _LL,
            1.0, 0.0).astype(jnp.bfloat16)
        counts = jnp.dot(obd, oneh.reshape(bb * _LL, _NB),
                         preferred_element_type=jnp.float32)
        hr = jnp.dot(counts * (1.0 / _LL), hcat[...])
        soft = jnp.dot(hr + lemean, wsp[...]) + bsp[...]

        x = xs.reshape(n_tok, _D)
        bf = jnp.bfloat16
        x = x.astype(bf)
        hmask = jnp.where(
            lax.broadcasted_iota(jnp.int32, (_NH * _LS, _D), 0) // _LS
            == lax.broadcasted_iota(jnp.int32, (_NH * _LS, _D), 1) // _DH,
            1.0, 0.0).astype(bf)

        for (wqkv, bqkv, wo, bo, g1, b1, w1, bff1, w2, bff2, g2, b2) in (lyr0, lyr1):
            qkv = (jnp.dot(x, wqkv[...].astype(bf), preferred_element_type=jnp.float32)
                   + bqkv[...]).astype(bf)
            vbigs, scs = [], []
            for smp in range(bb):
                sl = slice(smp * _LS, (smp + 1) * _LS)
                q_s = qkv[sl, :_D]
                k_s = qkv[sl, _D:2 * _D]
                v_s = qkv[sl, 2 * _D:]
                kbig = jnp.concatenate([k_s] * _NH, axis=0) * hmask
                vbigs.append(jnp.concatenate([v_s] * _NH, axis=0) * hmask)
                scs.append(lax.dot_general(
                    q_s, kbig, (((1,), (1,)), ((), ())),
                    preferred_element_type=jnp.float32).astype(bf))
            sall = jnp.concatenate(scs, axis=0) * bf(0.25)
            aparts = []
            for h in range(_NH):
                blk = sall[:, h * _LS:(h + 1) * _LS]
                bm = jnp.max(blk, axis=1, keepdims=True)
                ebk = jnp.exp(blk - bm)
                aparts.append(ebk / jnp.sum(ebk, axis=1, keepdims=True))
            aall = jnp.concatenate(aparts, axis=1)
            outs = [
                jnp.dot(aall[smp * _LS:(smp + 1) * _LS], vbigs[smp],
                        preferred_element_type=jnp.float32).astype(bf)
                for smp in range(bb)
            ]
            o = (jnp.dot(jnp.concatenate(outs, axis=0),
                         wo[...].astype(bf),
                         preferred_element_type=jnp.float32).astype(bf)
                 + bo[...].astype(bf))
            x = _layernorm(x + o, g1[...].astype(bf), b1[...].astype(bf))
            ff = (jnp.dot(
                jax.nn.relu(
                    jnp.dot(x, w1[...].astype(bf),
                            preferred_element_type=jnp.float32).astype(bf)
                    + bff1[...].astype(bf)),
                w2[...].astype(bf),
                preferred_element_type=jnp.float32).astype(bf)
                + bff2[...].astype(bf))
            x = _layernorm(x + ff, g2[...].astype(bf), b2[...].astype(bf))

        bst = jnp.mean(x.astype(jnp.float32).reshape(bb, _LS, _D), axis=1)

        comb = jnp.concatenate([uf, hard, soft, bst], axis=1)
        hh = jax.nn.relu(jnp.dot(comb, wfa[...]) + bfa[...])
        hh = jax.nn.relu(jnp.dot(hh, wfb[...]) + bfb[...])
        out_ref[...] = jnp.dot(hh, wfc[...]) + bfc[...]

    return body


def _tc_forward(uf, le, xs, tr, idm, p, bb=32, interpret=False):
    row = lambda v: v.reshape(1, -1)
    hcat = jnp.concatenate([p['hash0'], p['hash1'], p['hash2'], p['hash3']], axis=1)
    wa1 = p['Wa1']
    lyrs = []
    for i in range(2):
        lyrs.append([
            p['Wqkv%d' % i], row(p['bqkv%d' % i]), p['Wo%d' % i], row(p['bo%d' % i]),
            row(p['g1_%d' % i]), row(p['b1_%d' % i]), p['W1_%d' % i], row(p['bff1_%d' % i]),
            p['W2_%d' % i], row(p['bff2_%d' % i]), row(p['g2_%d' % i]), row(p['b2_%d' % i]),
        ])
    weights = [
        p['Wt'], row(p['bt']),
        wa1[:_D], wa1[_D:2 * _D], wa1[2 * _D:], row(p['ba1']),
        p['Wa2'], row(p['ba2']),
        hcat, p['Wsp'], row(p['bsp']),
        lyrs[0], lyrs[1],
        p['Wfa'], row(p['bfa']), p['Wfb'], row(p['bfb']), p['Wfc'], row(p['bfc']),
    ]

    def wspec(w):
        nd = w.ndim
        return pl.BlockSpec(w.shape, lambda i, _n=nd: (0,) * _n)

    in_specs = [
        pl.BlockSpec((bb, _D), lambda i: (i, 0)),
        pl.BlockSpec((bb, _LL, _D), lambda i: (i, 0, 0)),
        pl.BlockSpec((bb, _LS, _D), lambda i: (i, 0, 0)),
        pl.BlockSpec((bb, _D), lambda i: (i, 0)),
        pl.BlockSpec((bb, _LL), lambda i: (i, 0)),
    ] + jax.tree.map(wspec, weights)

    out = pl.pallas_call(
        _make_tc_body(bb),
        grid=(_B // bb,),
        in_specs=in_specs,
        out_specs=pl.BlockSpec((bb, 1), lambda i: (i, 0)),
        out_shape=jax.ShapeDtypeStruct((_B, 1), jnp.float32),
        interpret=interpret,
    )(uf, le, xs, tr, idm, *weights)
    return out[:, 0]


def kernel(user_features, params, target_item_id, short_hist_ids, long_hist_ids):
    ids_all = jnp.concatenate([
        long_hist_ids.reshape(-1),
        short_hist_ids.reshape(-1),
        target_item_id,
        jnp.zeros((_PAD_TOT - _TOT,), jnp.int32),
    ]).astype(jnp.int32).reshape(_NW, _NCH, _CHUNK)
    rows = _sc_gather(params['table'], ids_all)
    le = rows[:_B * _LL].reshape(_B, _LL, _D)
    xs = rows[_B * _LL:_B * _LL + _B * _LS].reshape(_B, _LS, _D)
    tr = rows[_B * _LL + _B * _LS:_TOT]
    idm = (long_hist_ids % _NB).astype(jnp.int32)
    return _tc_forward(user_features, le, xs, tr, idm, params)
